# baseline (device time: 840774 ns/iter reference)
import jax
import jax.numpy as jnp
from jax import lax
import jax.experimental.pallas as pl
from jax.experimental.pallas import tpu as pltpu

N_DEV = 16


def _ag_body(x_ref, w_ref, gx_ref, gw_ref,
             sx_r, rx_r, sx_l, rx_l, sw_r, rw_r, sw_l, rw_l):
    my = lax.axis_index("i")
    left = lax.rem(my + N_DEV - 1, N_DEV)
    right = lax.rem(my + 1, N_DEV)

    barrier = pltpu.get_barrier_semaphore()
    pl.semaphore_signal(barrier, inc=1, device_id=(left,),
                        device_id_type=pl.DeviceIdType.MESH)
    pl.semaphore_signal(barrier, inc=1, device_id=(right,),
                        device_id_type=pl.DeviceIdType.MESH)
    pl.semaphore_wait(barrier, 2)

    m, k_per = x_ref.shape
    _, n = w_ref.shape
    mh = m // 2
    nh = n // 2

    gx_ref[:, pl.ds(my * k_per, k_per)] = x_ref[...]
    gw_ref[pl.ds(my * k_per, k_per), :] = w_ref[...]

    all_rdmas = []
    for h in range(N_DEV - 1):
        src_r = lax.rem(my - h + N_DEV, N_DEV)
        src_l = lax.rem(my + h, N_DEV)
        cr = pl.ds(src_r * k_per, k_per)
        cl = pl.ds(src_l * k_per, k_per)
        rdmas = [
            pltpu.make_async_remote_copy(
                src_ref=gx_ref.at[pl.ds(0, mh), cr],
                dst_ref=gx_ref.at[pl.ds(0, mh), cr],
                send_sem=sx_r.at[h], recv_sem=rx_r.at[h],
                device_id=(right,), device_id_type=pl.DeviceIdType.MESH),
            pltpu.make_async_remote_copy(
                src_ref=gx_ref.at[pl.ds(mh, mh), cl],
                dst_ref=gx_ref.at[pl.ds(mh, mh), cl],
                send_sem=sx_l.at[h], recv_sem=rx_l.at[h],
                device_id=(left,), device_id_type=pl.DeviceIdType.MESH),
            pltpu.make_async_remote_copy(
                src_ref=gw_ref.at[cr, pl.ds(0, nh)],
                dst_ref=gw_ref.at[cr, pl.ds(0, nh)],
                send_sem=sw_r.at[h], recv_sem=rw_r.at[h],
                device_id=(right,), device_id_type=pl.DeviceIdType.MESH),
            pltpu.make_async_remote_copy(
                src_ref=gw_ref.at[cl, pl.ds(nh, nh)],
                dst_ref=gw_ref.at[cl, pl.ds(nh, nh)],
                send_sem=sw_l.at[h], recv_sem=rw_l.at[h],
                device_id=(left,), device_id_type=pl.DeviceIdType.MESH),
        ]
        for r in rdmas:
            r.start()
        for r in rdmas:
            r.wait_recv()
        all_rdmas.extend(rdmas)

    for r in all_rdmas:
        r.wait_send()


def _allgather_inputs(x, w_mat):
    m, k_per = x.shape
    _, n = w_mat.shape
    return pl.pallas_call(
        _ag_body,
        out_shape=[
            jax.ShapeDtypeStruct((m, N_DEV * k_per), jnp.int8),
            jax.ShapeDtypeStruct((N_DEV * k_per, n), jnp.int8),
        ],
        in_specs=[
            pl.BlockSpec(memory_space=pltpu.VMEM),
            pl.BlockSpec(memory_space=pltpu.VMEM),
        ],
        out_specs=[
            pl.BlockSpec(memory_space=pltpu.VMEM),
            pl.BlockSpec(memory_space=pltpu.VMEM),
        ],
        scratch_shapes=[pltpu.SemaphoreType.DMA((N_DEV - 1,))] * 8,
        compiler_params=pltpu.CompilerParams(
            collective_id=0,
            vmem_limit_bytes=56 * 1024 * 1024,
        ),
    )(x, w_mat)


_BM, _BN, _BK = 512, 4096, 256


def _mm_body(x_ref, w_ref, out_ref, acc_ref):
    k = pl.program_id(2)

    @pl.when(k == 0)
    def _():
        acc_ref[...] = jnp.zeros_like(acc_ref)

    acc_ref[...] += jnp.dot(
        x_ref[...], w_ref[...], preferred_element_type=jnp.float32
    )

    @pl.when(k == pl.num_programs(2) - 1)
    def _():
        y = acc_ref[...]
        out_ref[...] = y * jax.nn.sigmoid(y)


def _matmul_silu(xb, wb):
    m, k = xb.shape
    _, n = wb.shape
    return pl.pallas_call(
        _mm_body,
        out_shape=jax.ShapeDtypeStruct((m, n), jnp.float32),
        grid=(m // _BM, n // _BN, k // _BK),
        in_specs=[
            pl.BlockSpec((_BM, _BK), lambda i, j, kk: (i, kk)),
            pl.BlockSpec((_BK, _BN), lambda i, j, kk: (kk, j)),
        ],
        out_specs=pl.BlockSpec((_BM, _BN), lambda i, j, kk: (i, j)),
        scratch_shapes=[pltpu.VMEM((_BM, _BN), jnp.float32)],
        compiler_params=pltpu.CompilerParams(
            dimension_semantics=("parallel", "parallel", "arbitrary"),
        ),
    )(xb, wb)


def kernel(x, w_mat, scale_x, scale_w):
    gx, gw = _allgather_inputs(x, w_mat)
    s = (scale_x * scale_w).astype(jnp.float32)[0]
    xb = (gx.astype(jnp.float32) * s).astype(jnp.bfloat16)
    wb = gw.astype(jnp.bfloat16)
    return _matmul_silu(xb, wb)


# device time: 695988 ns/iter; 1.2080x vs baseline; 1.2080x over previous
import jax
import jax.numpy as jnp
from jax import lax
import jax.experimental.pallas as pl
from jax.experimental.pallas import tpu as pltpu

N_DEV = 16


def _ag_body(x_ref, w_ref, gx_ref, gw_ref,
             sx_r, rx_r, sx_l, rx_l, sw_r, rw_r, sw_l, rw_l):
    my = lax.axis_index("i")
    left = lax.rem(my + N_DEV - 1, N_DEV)
    right = lax.rem(my + 1, N_DEV)

    barrier = pltpu.get_barrier_semaphore()
    pl.semaphore_signal(barrier, inc=1, device_id=(left,),
                        device_id_type=pl.DeviceIdType.MESH)
    pl.semaphore_signal(barrier, inc=1, device_id=(right,),
                        device_id_type=pl.DeviceIdType.MESH)
    pl.semaphore_wait(barrier, 2)

    m, k_per = x_ref.shape
    _, n = w_ref.shape
    mh = m // 2
    nh = n // 2

    gx_ref[:, pl.ds(my * k_per, k_per)] = x_ref[...]
    gw_ref[pl.ds(my * k_per, k_per), :] = w_ref[...]

    all_rdmas = []
    for h in range(N_DEV - 1):
        src_r = lax.rem(my - h + N_DEV, N_DEV)
        src_l = lax.rem(my + h, N_DEV)
        cr = pl.ds(src_r * k_per, k_per)
        cl = pl.ds(src_l * k_per, k_per)
        rdmas = [
            pltpu.make_async_remote_copy(
                src_ref=gx_ref.at[pl.ds(0, mh), cr],
                dst_ref=gx_ref.at[pl.ds(0, mh), cr],
                send_sem=sx_r.at[h], recv_sem=rx_r.at[h],
                device_id=(right,), device_id_type=pl.DeviceIdType.MESH),
            pltpu.make_async_remote_copy(
                src_ref=gx_ref.at[pl.ds(mh, mh), cl],
                dst_ref=gx_ref.at[pl.ds(mh, mh), cl],
                send_sem=sx_l.at[h], recv_sem=rx_l.at[h],
                device_id=(left,), device_id_type=pl.DeviceIdType.MESH),
            pltpu.make_async_remote_copy(
                src_ref=gw_ref.at[cr, pl.ds(0, nh)],
                dst_ref=gw_ref.at[cr, pl.ds(0, nh)],
                send_sem=sw_r.at[h], recv_sem=rw_r.at[h],
                device_id=(right,), device_id_type=pl.DeviceIdType.MESH),
            pltpu.make_async_remote_copy(
                src_ref=gw_ref.at[cl, pl.ds(nh, nh)],
                dst_ref=gw_ref.at[cl, pl.ds(nh, nh)],
                send_sem=sw_l.at[h], recv_sem=rw_l.at[h],
                device_id=(left,), device_id_type=pl.DeviceIdType.MESH),
        ]
        for r in rdmas:
            r.start()
        for r in rdmas:
            r.wait_recv()
        all_rdmas.extend(rdmas)

    for r in all_rdmas:
        r.wait_send()


def _allgather_inputs(x, w_mat):
    m, k_per = x.shape
    _, n = w_mat.shape
    return pl.pallas_call(
        _ag_body,
        out_shape=[
            jax.ShapeDtypeStruct((m, N_DEV * k_per), jnp.int8),
            jax.ShapeDtypeStruct((N_DEV * k_per, n), jnp.int8),
        ],
        in_specs=[
            pl.BlockSpec(memory_space=pltpu.VMEM),
            pl.BlockSpec(memory_space=pltpu.VMEM),
        ],
        out_specs=[
            pl.BlockSpec(memory_space=pltpu.VMEM),
            pl.BlockSpec(memory_space=pltpu.VMEM),
        ],
        scratch_shapes=[pltpu.SemaphoreType.DMA((N_DEV - 1,))] * 8,
        compiler_params=pltpu.CompilerParams(
            collective_id=0,
            vmem_limit_bytes=56 * 1024 * 1024,
        ),
    )(x, w_mat)


_BM, _BN = 512, 1024


def _mm_body(x_ref, w_ref, out_ref):
    y = jnp.dot(x_ref[...], w_ref[...], preferred_element_type=jnp.float32)
    out_ref[...] = y * jax.nn.sigmoid(y)


def _matmul_silu(xb, wb):
    m, k = xb.shape
    _, n = wb.shape
    return pl.pallas_call(
        _mm_body,
        out_shape=jax.ShapeDtypeStruct((m, n), jnp.float32),
        grid=(m // _BM, n // _BN),
        in_specs=[
            pl.BlockSpec((_BM, k), lambda i, j: (i, 0)),
            pl.BlockSpec((k, _BN), lambda i, j: (0, j)),
        ],
        out_specs=pl.BlockSpec((_BM, _BN), lambda i, j: (i, j)),
        compiler_params=pltpu.CompilerParams(
            dimension_semantics=("parallel", "parallel"),
            vmem_limit_bytes=48 * 1024 * 1024,
        ),
    )(xb, wb)


def kernel(x, w_mat, scale_x, scale_w):
    gx, gw = _allgather_inputs(x, w_mat)
    s = (scale_x * scale_w).astype(jnp.float32)[0]
    xb = (gx.astype(jnp.float32) * s).astype(jnp.bfloat16)
    wb = gw.astype(jnp.bfloat16)
    return _matmul_silu(xb, wb)


# device time: 633045 ns/iter; 1.3281x vs baseline; 1.0994x over previous
import jax
import jax.numpy as jnp
from jax import lax
import jax.experimental.pallas as pl
from jax.experimental.pallas import tpu as pltpu

N_DEV = 16


def _ag_body(x_ref, w_ref, s_ref, gxb_ref, gwb_ref, gx, gw,
             sx_r, rx_r, sx_l, rx_l, sw_r, rw_r, sw_l, rw_l,
             st_xt, st_xb, st_wl, st_wr, dma_sems):
    my = lax.axis_index("i")
    left = lax.rem(my + N_DEV - 1, N_DEV)
    right = lax.rem(my + 1, N_DEV)

    barrier = pltpu.get_barrier_semaphore()
    pl.semaphore_signal(barrier, inc=1, device_id=(left,),
                        device_id_type=pl.DeviceIdType.MESH)
    pl.semaphore_signal(barrier, inc=1, device_id=(right,),
                        device_id_type=pl.DeviceIdType.MESH)
    pl.semaphore_wait(barrier, 2)

    m, k_per = x_ref.shape
    _, n = w_ref.shape
    mh = m // 2
    nh = n // 2
    s = s_ref[0, 0]

    gx[:, pl.ds(my * k_per, k_per)] = x_ref[...]
    gw[pl.ds(my * k_per, k_per), :] = w_ref[...]

    def start_hop(h):
        src_r = lax.rem(my - h + N_DEV, N_DEV)
        src_l = lax.rem(my + h, N_DEV)
        cr = pl.ds(src_r * k_per, k_per)
        cl = pl.ds(src_l * k_per, k_per)
        rdmas = [
            pltpu.make_async_remote_copy(
                src_ref=gx.at[pl.ds(0, mh), cr],
                dst_ref=gx.at[pl.ds(0, mh), cr],
                send_sem=sx_r.at[h], recv_sem=rx_r.at[h],
                device_id=(right,), device_id_type=pl.DeviceIdType.MESH),
            pltpu.make_async_remote_copy(
                src_ref=gx.at[pl.ds(mh, mh), cl],
                dst_ref=gx.at[pl.ds(mh, mh), cl],
                send_sem=sx_l.at[h], recv_sem=rx_l.at[h],
                device_id=(left,), device_id_type=pl.DeviceIdType.MESH),
            pltpu.make_async_remote_copy(
                src_ref=gw.at[cr, pl.ds(0, nh)],
                dst_ref=gw.at[cr, pl.ds(0, nh)],
                send_sem=sw_r.at[h], recv_sem=rw_r.at[h],
                device_id=(right,), device_id_type=pl.DeviceIdType.MESH),
            pltpu.make_async_remote_copy(
                src_ref=gw.at[cl, pl.ds(nh, nh)],
                dst_ref=gw.at[cl, pl.ds(nh, nh)],
                send_sem=sw_l.at[h], recv_sem=rw_l.at[h],
                device_id=(left,), device_id_type=pl.DeviceIdType.MESH),
        ]
        for r in rdmas:
            r.start()
        return rdmas

    pending = {}

    def flush(stage, sem_i, value, dst):
        if sem_i in pending:
            pending[sem_i].wait()
        stage[...] = value
        d = pltpu.make_async_copy(stage, dst, dma_sems.at[sem_i])
        d.start()
        pending[sem_i] = d

    def cast_pieces(origin, top, bot, wl, wr):
        c = pl.ds(origin * k_per, k_per)
        if top:
            flush(st_xt, 0,
                  (gx[pl.ds(0, mh), c].astype(jnp.float32) * s
                   ).astype(jnp.bfloat16),
                  gxb_ref.at[pl.ds(0, mh), c])
        if bot:
            flush(st_xb, 1,
                  (gx[pl.ds(mh, mh), c].astype(jnp.float32) * s
                   ).astype(jnp.bfloat16),
                  gxb_ref.at[pl.ds(mh, mh), c])
        if wl:
            flush(st_wl, 2, gw[c, pl.ds(0, nh)].astype(jnp.bfloat16),
                  gwb_ref.at[c, pl.ds(0, nh)])
        if wr:
            flush(st_wr, 3, gw[c, pl.ds(nh, nh)].astype(jnp.bfloat16),
                  gwb_ref.at[c, pl.ds(nh, nh)])

    hop_rdmas = start_hop(0)
    cast_pieces(my, True, True, True, True)
    all_rdmas = list(hop_rdmas)
    for h in range(N_DEV - 1):
        for r in hop_rdmas:
            r.wait_recv()
        if h + 1 < N_DEV - 1:
            hop_rdmas = start_hop(h + 1)
            all_rdmas.extend(hop_rdmas)
        o_r = lax.rem(my - 1 - h + N_DEV, N_DEV)
        o_l = lax.rem(my + 1 + h, N_DEV)
        cast_pieces(o_r, True, False, True, False)
        cast_pieces(o_l, False, True, False, True)

    for d in pending.values():
        d.wait()
    for r in all_rdmas:
        r.wait_send()


def _allgather_cast(x, w_mat, s):
    m, k_per = x.shape
    _, n = w_mat.shape
    return pl.pallas_call(
        _ag_body,
        out_shape=[
            jax.ShapeDtypeStruct((m, N_DEV * k_per), jnp.bfloat16),
            jax.ShapeDtypeStruct((N_DEV * k_per, n), jnp.bfloat16),
        ],
        in_specs=[
            pl.BlockSpec(memory_space=pltpu.VMEM),
            pl.BlockSpec(memory_space=pltpu.VMEM),
            pl.BlockSpec(memory_space=pltpu.VMEM),
        ],
        out_specs=[
            pl.BlockSpec(memory_space=pltpu.MemorySpace.HBM),
            pl.BlockSpec(memory_space=pltpu.MemorySpace.HBM),
        ],
        scratch_shapes=[
            pltpu.VMEM((m, N_DEV * k_per), jnp.int8),
            pltpu.VMEM((N_DEV * k_per, n), jnp.int8),
        ] + [pltpu.SemaphoreType.DMA((N_DEV - 1,))] * 8 + [
            pltpu.VMEM((m // 2, k_per), jnp.bfloat16),
            pltpu.VMEM((m // 2, k_per), jnp.bfloat16),
            pltpu.VMEM((k_per, n // 2), jnp.bfloat16),
            pltpu.VMEM((k_per, n // 2), jnp.bfloat16),
            pltpu.SemaphoreType.DMA((4,)),
        ],
        compiler_params=pltpu.CompilerParams(
            collective_id=0,
            vmem_limit_bytes=60 * 1024 * 1024,
        ),
    )(x, w_mat, s)


_BM, _BN = 512, 1024


def _mm_body(x_ref, w_ref, out_ref):
    y = jnp.dot(x_ref[...], w_ref[...], preferred_element_type=jnp.float32)
    out_ref[...] = y * jax.nn.sigmoid(y)


def _matmul_silu(xb, wb):
    m, k = xb.shape
    _, n = wb.shape
    return pl.pallas_call(
        _mm_body,
        out_shape=jax.ShapeDtypeStruct((m, n), jnp.float32),
        grid=(m // _BM, n // _BN),
        in_specs=[
            pl.BlockSpec((_BM, k), lambda i, j: (i, 0)),
            pl.BlockSpec((k, _BN), lambda i, j: (0, j)),
        ],
        out_specs=pl.BlockSpec((_BM, _BN), lambda i, j: (i, j)),
        compiler_params=pltpu.CompilerParams(
            dimension_semantics=("parallel", "parallel"),
            vmem_limit_bytes=48 * 1024 * 1024,
        ),
    )(xb, wb)


def kernel(x, w_mat, scale_x, scale_w):
    s = jnp.reshape((scale_x * scale_w).astype(jnp.float32), (1, 1))
    gxb, gwb = _allgather_cast(x, w_mat, s)
    return _matmul_silu(gxb, gwb)


# device time: 618187 ns/iter; 1.3601x vs baseline; 1.0240x over previous
import jax
import jax.numpy as jnp
from jax import lax
import jax.experimental.pallas as pl
from jax.experimental.pallas import tpu as pltpu

N_DEV = 16


def _ag_body(x_ref, w_ref, s_ref, gxb_ref, gwb_ref, gx, gw,
             sx_r, rx_r, sx_l, rx_l, sw_r, rw_r, sw_l, rw_l,
             st_xt, st_xb, st_wl, st_wr, dma_sems):
    my = lax.axis_index("i")
    left = lax.rem(my + N_DEV - 1, N_DEV)
    right = lax.rem(my + 1, N_DEV)

    barrier = pltpu.get_barrier_semaphore()
    pl.semaphore_signal(barrier, inc=1, device_id=(left,),
                        device_id_type=pl.DeviceIdType.MESH)
    pl.semaphore_signal(barrier, inc=1, device_id=(right,),
                        device_id_type=pl.DeviceIdType.MESH)
    pl.semaphore_wait(barrier, 2)

    m, k_per = x_ref.shape
    _, n = w_ref.shape
    mh = m // 2
    nh = n // 2
    s = s_ref[0, 0]

    gx[:, pl.ds(my * k_per, k_per)] = x_ref[...]
    gw[pl.ds(my * k_per, k_per), :] = w_ref[...]

    def start_hop(h):
        src_r = lax.rem(my - h + N_DEV, N_DEV)
        src_l = lax.rem(my + h, N_DEV)
        cr = pl.ds(src_r * k_per, k_per)
        cl = pl.ds(src_l * k_per, k_per)
        rdmas = [
            pltpu.make_async_remote_copy(
                src_ref=gx.at[pl.ds(0, mh), cr],
                dst_ref=gx.at[pl.ds(0, mh), cr],
                send_sem=sx_r.at[h], recv_sem=rx_r.at[h],
                device_id=(right,), device_id_type=pl.DeviceIdType.MESH),
            pltpu.make_async_remote_copy(
                src_ref=gx.at[pl.ds(mh, mh), cl],
                dst_ref=gx.at[pl.ds(mh, mh), cl],
                send_sem=sx_l.at[h], recv_sem=rx_l.at[h],
                device_id=(left,), device_id_type=pl.DeviceIdType.MESH),
            pltpu.make_async_remote_copy(
                src_ref=gw.at[cr, pl.ds(0, nh)],
                dst_ref=gw.at[cr, pl.ds(0, nh)],
                send_sem=sw_r.at[h], recv_sem=rw_r.at[h],
                device_id=(right,), device_id_type=pl.DeviceIdType.MESH),
            pltpu.make_async_remote_copy(
                src_ref=gw.at[cl, pl.ds(nh, nh)],
                dst_ref=gw.at[cl, pl.ds(nh, nh)],
                send_sem=sw_l.at[h], recv_sem=rw_l.at[h],
                device_id=(left,), device_id_type=pl.DeviceIdType.MESH),
        ]
        for r in rdmas:
            r.start()
        return rdmas

    pending = {}

    def flush(stage, sem_i, value, dst):
        if sem_i in pending:
            pending[sem_i].wait()
        stage[...] = value
        d = pltpu.make_async_copy(stage, dst, dma_sems.at[sem_i])
        d.start()
        pending[sem_i] = d

    def cast_pieces(origin, top, bot, wl, wr):
        c = pl.ds(origin * k_per, k_per)
        if top:
            flush(st_xt, 0,
                  (gx[pl.ds(0, mh), c].astype(jnp.float32) * s
                   ).astype(jnp.bfloat16),
                  gxb_ref.at[pl.ds(0, mh), c])
        if bot:
            flush(st_xb, 1,
                  (gx[pl.ds(mh, mh), c].astype(jnp.float32) * s
                   ).astype(jnp.bfloat16),
                  gxb_ref.at[pl.ds(mh, mh), c])
        if wl:
            flush(st_wl, 2, gw[c, pl.ds(0, nh)].astype(jnp.bfloat16),
                  gwb_ref.at[c, pl.ds(0, nh)])
        if wr:
            flush(st_wr, 3, gw[c, pl.ds(nh, nh)].astype(jnp.bfloat16),
                  gwb_ref.at[c, pl.ds(nh, nh)])

    hop_rdmas = start_hop(0)
    cast_pieces(my, True, True, True, True)
    all_rdmas = list(hop_rdmas)
    for h in range(N_DEV - 1):
        for r in hop_rdmas:
            r.wait_recv()
        if h + 1 < N_DEV - 1:
            hop_rdmas = start_hop(h + 1)
            all_rdmas.extend(hop_rdmas)
        o_r = lax.rem(my - 1 - h + N_DEV, N_DEV)
        o_l = lax.rem(my + 1 + h, N_DEV)
        cast_pieces(o_r, True, False, True, False)
        cast_pieces(o_l, False, True, False, True)

    for d in pending.values():
        d.wait()
    for r in all_rdmas:
        r.wait_send()


def _allgather_cast(x, w_mat, s):
    m, k_per = x.shape
    _, n = w_mat.shape
    return pl.pallas_call(
        _ag_body,
        out_shape=[
            jax.ShapeDtypeStruct((m, N_DEV * k_per), jnp.bfloat16),
            jax.ShapeDtypeStruct((N_DEV * k_per, n), jnp.bfloat16),
        ],
        in_specs=[
            pl.BlockSpec(memory_space=pltpu.VMEM),
            pl.BlockSpec(memory_space=pltpu.VMEM),
            pl.BlockSpec(memory_space=pltpu.VMEM),
        ],
        out_specs=[
            pl.BlockSpec(memory_space=pltpu.MemorySpace.HBM),
            pl.BlockSpec(memory_space=pltpu.MemorySpace.HBM),
        ],
        scratch_shapes=[
            pltpu.VMEM((m, N_DEV * k_per), jnp.int8),
            pltpu.VMEM((N_DEV * k_per, n), jnp.int8),
        ] + [pltpu.SemaphoreType.DMA((N_DEV - 1,))] * 8 + [
            pltpu.VMEM((m // 2, k_per), jnp.bfloat16),
            pltpu.VMEM((m // 2, k_per), jnp.bfloat16),
            pltpu.VMEM((k_per, n // 2), jnp.bfloat16),
            pltpu.VMEM((k_per, n // 2), jnp.bfloat16),
            pltpu.SemaphoreType.DMA((4,)),
        ],
        compiler_params=pltpu.CompilerParams(
            collective_id=0,
            vmem_limit_bytes=60 * 1024 * 1024,
        ),
    )(x, w_mat, s)


_BM, _BN = 1024, 1024


def _mm_body(x_ref, w_ref, out_ref):
    y = jnp.dot(x_ref[...], w_ref[...], preferred_element_type=jnp.float32)
    out_ref[...] = y * jax.nn.sigmoid(y)


def _matmul_silu(xb, wb):
    m, k = xb.shape
    _, n = wb.shape
    return pl.pallas_call(
        _mm_body,
        out_shape=jax.ShapeDtypeStruct((m, n), jnp.float32),
        grid=(m // _BM, n // _BN),
        in_specs=[
            pl.BlockSpec((_BM, k), lambda i, j: (i, 0)),
            pl.BlockSpec((k, _BN), lambda i, j: (0, j)),
        ],
        out_specs=pl.BlockSpec((_BM, _BN), lambda i, j: (i, j)),
        compiler_params=pltpu.CompilerParams(
            dimension_semantics=("parallel", "parallel"),
            vmem_limit_bytes=56 * 1024 * 1024,
        ),
    )(xb, wb)


def kernel(x, w_mat, scale_x, scale_w):
    s = jnp.reshape((scale_x * scale_w).astype(jnp.float32), (1, 1))
    gxb, gwb = _allgather_cast(x, w_mat, s)
    return _matmul_silu(gxb, gwb)
